# Initial kernel scaffold; baseline (speedup 1.0000x reference)
import jax
import jax.numpy as jnp
from jax.experimental import pallas as pl


def kernel(x, gate_w, gate_b, gammas, betas):
    raise NotImplementedError("write your pallas kernel here")


# speedup vs baseline: 7.1312x; 7.1312x over previous
import jax
import jax.numpy as jnp
from jax.experimental import pallas as pl
from jax.experimental.pallas import tpu as pltpu

NUM_EXPERTS = 16
HID = 1024
B_, C_, T_ = 2, 1024, 4096
TCB = 512
NT = T_ // TCB
LANES = 128
KSUB = TCB // LANES


def _body(x_ref, gw_ref, gb_ref, gam_ref, bet_ref, out_ref,
          stash_ref, acc_ref, gsel_ref, bsel_ref):
    p = pl.program_id(0)
    j = pl.program_id(1)

    @pl.when(p == 0)
    def _reduce():
        xb = x_ref[...]
        stash_ref[:, :, pl.ds(j * TCB, TCB)] = xb
        s = xb[:, :, 0 * LANES:1 * LANES]
        for k in range(1, KSUB):
            s = s + xb[:, :, k * LANES:(k + 1) * LANES]

        @pl.when(j == 0)
        def _():
            acc_ref[...] = s

        @pl.when(j > 0)
        def _():
            acc_ref[...] = acc_ref[...] + s

        @pl.when(j == NT - 1)
        def _gate():
            gate_input = jnp.sum(acc_ref[...], axis=-1) * (1.0 / T_)
            scores = jax.lax.dot_general(
                gate_input, gw_ref[...],
                (((1,), (1,)), ((), ())),
                preferred_element_type=jnp.float32,
            ) + gb_ref[...]
            m = jnp.max(scores, axis=-1, keepdims=True)
            iota = jax.lax.broadcasted_iota(jnp.int32, (B_, NUM_EXPERTS), 1)
            idx = jnp.min(
                jnp.where(scores >= m, iota, NUM_EXPERTS),
                axis=-1, keepdims=True)
            sel = iota == idx
            gsel_ref[...] = jnp.sum(
                jnp.where(sel, gam_ref[...], 0.0), axis=-1, keepdims=True)
            bsel_ref[...] = jnp.sum(
                jnp.where(sel, bet_ref[...], 0.0), axis=-1, keepdims=True)

    @pl.when(p == 1)
    def _apply():
        g = gsel_ref[...][:, :, None]
        b = bsel_ref[...][:, :, None]
        out_ref[...] = stash_ref[:, :, pl.ds(j * TCB, TCB)] * g + b


def kernel(x, gate_w, gate_b, gammas, betas):
    xs = x.reshape(B_, C_, T_)
    out = pl.pallas_call(
        _body,
        grid=(2, NT),
        in_specs=[
            pl.BlockSpec((B_, C_, TCB),
                         lambda p, j: (0, 0, jnp.where(p == 0, j, NT - 1))),
            pl.BlockSpec((NUM_EXPERTS, HID), lambda p, j: (0, 0)),
            pl.BlockSpec((1, NUM_EXPERTS), lambda p, j: (0, 0)),
            pl.BlockSpec((1, NUM_EXPERTS), lambda p, j: (0, 0)),
            pl.BlockSpec((1, NUM_EXPERTS), lambda p, j: (0, 0)),
        ],
        out_specs=pl.BlockSpec(
            (B_, C_, TCB), lambda p, j: (0, 0, jnp.where(p == 0, 0, j))),
        out_shape=jax.ShapeDtypeStruct((B_, C_, T_), jnp.float32),
        scratch_shapes=[
            pltpu.VMEM((B_, C_, T_), jnp.float32),
            pltpu.VMEM((B_, C_, LANES), jnp.float32),
            pltpu.VMEM((B_, 1), jnp.float32),
            pltpu.VMEM((B_, 1), jnp.float32),
        ],
        compiler_params=pltpu.CompilerParams(
            dimension_semantics=("arbitrary", "arbitrary")),
    )(xs, gate_w, gate_b.reshape(1, NUM_EXPERTS),
      gammas.reshape(1, NUM_EXPERTS), betas.reshape(1, NUM_EXPERTS))
    return out.reshape(B_, C_, T_, 1)


# speedup vs baseline: 23.5585x; 3.3036x over previous
import jax
import jax.numpy as jnp
from jax.experimental import pallas as pl
from jax.experimental.pallas import tpu as pltpu

NUM_EXPERTS = 16
HID = 1024
B_, C_, T_ = 2, 1024, 4096
LANES = 128
TH = T_ // LANES
RTOT = B_ * C_
RB = 256
NT = RTOT // RB
NB0 = C_ // RB


def _body(x_ref, gw_ref, gb_ref, gam_ref, bet_ref, out_ref,
          stash_ref, acc_ref, gsel_ref, bsel_ref):
    p = pl.program_id(0)
    j = pl.program_id(1)

    @pl.when(p == 0)
    def _reduce():
        xb = x_ref[...]
        stash_ref[pl.ds(j * RB, RB)] = xb
        s = xb[:, 0:8, :]
        for k in range(1, TH // 8):
            s = s + xb[:, 8 * k:8 * (k + 1), :]
        acc_ref[pl.ds(j * RB, RB)] = jnp.sum(s, axis=1)

        @pl.when(j == NT - 1)
        def _gate():
            gi = acc_ref[...]
            iota = jax.lax.broadcasted_iota(
                jnp.int32, (NUM_EXPERTS, 1), 0)
            for b in range(B_):
                pb = jax.lax.dot_general(
                    gw_ref[...], gi[b * C_:(b + 1) * C_, :],
                    (((1,), (0,)), ((), ())),
                    preferred_element_type=jnp.float32,
                )
                scores = (jnp.sum(pb, axis=-1, keepdims=True) * (1.0 / T_)
                          + gb_ref[...])
                m = jnp.max(scores, axis=0, keepdims=True)
                idx = jnp.min(
                    jnp.where(scores >= m, iota, NUM_EXPERTS),
                    axis=0, keepdims=True)
                sel = iota == idx
                gsel_ref[b:b + 1, :] = jnp.sum(
                    jnp.where(sel, gam_ref[...], 0.0), axis=0, keepdims=True)
                bsel_ref[b:b + 1, :] = jnp.sum(
                    jnp.where(sel, bet_ref[...], 0.0), axis=0, keepdims=True)

    @pl.when(p == 1)
    def _apply():
        gsel = gsel_ref[...]
        bsel = bsel_ref[...]
        g = jnp.where(j < NB0, gsel[0:1, 0:1], gsel[1:2, 0:1])
        b = jnp.where(j < NB0, bsel[0:1, 0:1], bsel[1:2, 0:1])
        out_ref[...] = (stash_ref[pl.ds(j * RB, RB)] * g[:, :, None]
                        + b[:, :, None])


def kernel(x, gate_w, gate_b, gammas, betas):
    xs = x.reshape(RTOT, TH, LANES)
    out = pl.pallas_call(
        _body,
        grid=(2, NT),
        in_specs=[
            pl.BlockSpec((RB, TH, LANES),
                         lambda p, j: (jnp.where(p == 0, j, NT - 1), 0, 0)),
            pl.BlockSpec((NUM_EXPERTS, HID), lambda p, j: (0, 0)),
            pl.BlockSpec((NUM_EXPERTS, 1), lambda p, j: (0, 0)),
            pl.BlockSpec((NUM_EXPERTS, 1), lambda p, j: (0, 0)),
            pl.BlockSpec((NUM_EXPERTS, 1), lambda p, j: (0, 0)),
        ],
        out_specs=pl.BlockSpec(
            (RB, TH, LANES), lambda p, j: (jnp.where(p == 0, 0, j), 0, 0)),
        out_shape=jax.ShapeDtypeStruct((RTOT, TH, LANES), jnp.float32),
        scratch_shapes=[
            pltpu.VMEM((RTOT, TH, LANES), jnp.float32),
            pltpu.VMEM((RTOT, LANES), jnp.float32),
            pltpu.VMEM((B_, 1), jnp.float32),
            pltpu.VMEM((B_, 1), jnp.float32),
        ],
        compiler_params=pltpu.CompilerParams(
            dimension_semantics=("arbitrary", "arbitrary")),
    )(xs, gate_w, gate_b.reshape(NUM_EXPERTS, 1),
      gammas.reshape(NUM_EXPERTS, 1), betas.reshape(NUM_EXPERTS, 1))
    return out.reshape(B_, C_, T_, 1)
